# Initial kernel scaffold; baseline (speedup 1.0000x reference)
import jax
import jax.numpy as jnp
from jax.experimental import pallas as pl


def kernel(x, edge_index, W1, W2, W3, W4, W5, W6, W7, W8, b1, b2, b3, b4, b5, b6, b7, b8):
    raise NotImplementedError("write your pallas kernel here")


# speedup vs baseline: 13.4261x; 13.4261x over previous
import functools

import jax
import jax.numpy as jnp
from jax import lax
from jax.experimental import pallas as pl
from jax.experimental.pallas import tpu as pltpu
from jax.experimental.pallas import tpu_sc as plsc

N = 10000
E = 320000
NC = 2
NS = 16
NW = NC * NS
EPW = E // NW
G = 80
NCHUNK = EPW // G
NP = 10112
RPS = NP // NS


def _propagate(u, src3, dst3, zeros, d):
    mesh = plsc.VectorSubcoreMesh(core_axis_name="c", subcore_axis_name="s")

    @functools.partial(
        pl.kernel,
        mesh=mesh,
        out_type=jax.ShapeDtypeStruct((NC, NP, d), jnp.float32),
        scratch_types=[
            pltpu.VMEM((NCHUNK, G), jnp.int32),
            pltpu.VMEM((NCHUNK, G), jnp.int32),
            pltpu.VMEM((G, d), jnp.float32),
            pltpu.VMEM_SHARED((NP, d), jnp.float32),
            pltpu.SemaphoreType.DMA,
        ],
        compiler_params=pltpu.CompilerParams(use_tc_tiling_on_sc=False),
    )
    def k(u_hbm, src_hbm, dst_hbm, zero_hbm, out_hbm, src_v, dst_v, rows_v, acc, sem):
        c = lax.axis_index("c")
        s = lax.axis_index("s")
        wid = c * NS + s

        pltpu.sync_copy(zero_hbm, acc.at[pl.ds(s * RPS, RPS)])
        pltpu.sync_copy(src_hbm.at[wid], src_v)
        pltpu.sync_copy(dst_hbm.at[wid], dst_v)
        plsc.subcore_barrier()

        @pl.loop(0, NCHUNK)
        def _(j):
            pltpu.async_copy(u_hbm.at[src_v.at[j]], rows_v, sem).wait()
            pltpu.sync_copy(rows_v, acc.at[dst_v.at[j]], add=True)

        plsc.subcore_barrier()
        pltpu.sync_copy(
            acc.at[pl.ds(s * RPS, RPS)], out_hbm.at[c, pl.ds(s * RPS, RPS)]
        )

    return k(u, src3, dst3, zeros)


def _first_layer(x, W1, deg2):
    def body(deg2_ref, x_ref, w_ref, g_ref, u_ref, dinv_ref, dinv2_ref):
        deg = deg2_ref[0, 0:N, 0:1] + deg2_ref[1, 0:N, 0:1] + 1.0
        dinv = lax.rsqrt(deg)
        dinv_ref[...] = dinv
        dinv2_ref[...] = dinv * dinv
        g = jnp.dot(x_ref[...], w_ref[...], preferred_element_type=jnp.float32)
        g_ref[...] = g
        u_ref[...] = dinv * g

    dout = W1.shape[1]
    return pl.pallas_call(
        body,
        out_shape=(
            jax.ShapeDtypeStruct((N, dout), jnp.float32),
            jax.ShapeDtypeStruct((N, dout), jnp.float32),
            jax.ShapeDtypeStruct((N, 1), jnp.float32),
            jax.ShapeDtypeStruct((N, 1), jnp.float32),
        ),
    )(deg2, x, W1)


def _mid_layer(s2, g, dinv, dinv2, b, W, relu, pad_u_to=None):
    def body(s2_ref, g_ref, dinv_ref, dinv2_ref, b_ref, w_ref, g_out, u_out):
        pre = (
            dinv_ref[...] * (s2_ref[0, 0:N] + s2_ref[1, 0:N])
            + dinv2_ref[...] * g_ref[...]
            + b_ref[...]
        )
        h = jnp.maximum(pre, 0.0) if relu else pre
        gn = jnp.dot(h, w_ref[...], preferred_element_type=jnp.float32)
        g_out[...] = gn
        un = dinv_ref[...] * gn
        if pad_u_to is None:
            u_out[...] = un
        else:
            col = lax.broadcasted_iota(jnp.int32, (N, pad_u_to), 1)
            u_out[...] = jnp.where(col == 0, un, 0.0)

    dout = W.shape[1]
    du = dout if pad_u_to is None else pad_u_to
    return pl.pallas_call(
        body,
        out_shape=(
            jax.ShapeDtypeStruct((N, dout), jnp.float32),
            jax.ShapeDtypeStruct((N, du), jnp.float32),
        ),
    )(s2, g, dinv, dinv2, b.reshape(1, W.shape[0]), W)


def _final_layer(s2, g, dinv, dinv2, b):
    def body(s2_ref, g_ref, dinv_ref, dinv2_ref, b_ref, o_ref):
        srow = s2_ref[0, 0:N, 0:1] + s2_ref[1, 0:N, 0:1]
        o_ref[...] = dinv_ref[...] * srow + dinv2_ref[...] * g_ref[...] + b_ref[...]

    return pl.pallas_call(
        body,
        out_shape=jax.ShapeDtypeStruct((N, 1), jnp.float32),
    )(s2, g, dinv, dinv2, b.reshape(1, 1))


def kernel(x, edge_index, W1, W2, W3, W4, W5, W6, W7, W8, b1, b2, b3, b4, b5, b6, b7, b8):
    src3 = edge_index[0].reshape(NW, NCHUNK, G)
    dst3 = edge_index[1].reshape(NW, NCHUNK, G)

    ones16 = jnp.ones((N, 16), jnp.float32)
    zeros16 = jnp.zeros((RPS, 16), jnp.float32)

    deg2 = _propagate(ones16, src3, dst3, zeros16, 16)
    g, u, dinv, dinv2 = _first_layer(x, W1, deg2)

    Ws = [W2, W3, W4, W5, W6, W7, W8]
    bs = [b1, b2, b3, b4, b5, b6, b7]
    relus = [True, True, True, False, True, True, True]
    for i in range(7):
        d = u.shape[1]
        s2 = _propagate(u, src3, dst3, jnp.zeros((RPS, d), jnp.float32), d)
        pad = 16 if i == 6 else None
        g, u = _mid_layer(s2, g, dinv, dinv2, bs[i], Ws[i], relus[i], pad_u_to=pad)

    s2 = _propagate(u, src3, dst3, zeros16, 16)
    return _final_layer(s2, g, dinv, dinv2, b8)


# speedup vs baseline: 21.2359x; 1.5817x over previous
import functools

import jax
import jax.numpy as jnp
from jax import lax
from jax.experimental import pallas as pl
from jax.experimental.pallas import tpu as pltpu
from jax.experimental.pallas import tpu_sc as plsc

N = 10000
E = 320000
NC = 2
NS = 16
NW = NC * NS
EPW = E // NW
G = 80
NCHUNK = EPW // G
NP = 10112
RPS = NP // NS


def _propagate(u, src3, dst3, zeros, d):
    mesh = plsc.VectorSubcoreMesh(core_axis_name="c", subcore_axis_name="s")

    @functools.partial(
        pl.kernel,
        mesh=mesh,
        out_type=jax.ShapeDtypeStruct((NC, NP, d), jnp.float32),
        scratch_types=[
            pltpu.VMEM((NCHUNK, G), jnp.int32),
            pltpu.VMEM((NCHUNK, G), jnp.int32),
            pltpu.VMEM((G, d), jnp.float32),
            pltpu.VMEM((G, d), jnp.float32),
            pltpu.VMEM_SHARED((NP, d), jnp.float32),
            pltpu.SemaphoreType.DMA,
            pltpu.SemaphoreType.DMA,
        ],
        compiler_params=pltpu.CompilerParams(use_tc_tiling_on_sc=False),
    )
    def k(u_hbm, src_hbm, dst_hbm, zero_hbm, out_hbm, src_v, dst_v, rows_a, rows_b, acc, sem_a, sem_b):
        c = lax.axis_index("c")
        s = lax.axis_index("s")
        wid = c * NS + s

        pltpu.sync_copy(zero_hbm, acc.at[pl.ds(s * RPS, RPS)])
        pltpu.sync_copy(src_hbm.at[wid], src_v)
        pltpu.sync_copy(dst_hbm.at[wid], dst_v)
        plsc.subcore_barrier()

        def fire(j, buf, sem):
            pltpu.async_copy(u_hbm.at[src_v.at[j]], buf, sem)

        def wait(buf, sem):
            pltpu.make_async_copy(u_hbm.at[src_v.at[0]], buf, sem).wait()

        fire(0, rows_a, sem_a)

        @pl.loop(0, (NCHUNK - 1) // 2)
        def _(i):
            j = 2 * i
            fire(j + 1, rows_b, sem_b)
            wait(rows_a, sem_a)
            pltpu.sync_copy(rows_a, acc.at[dst_v.at[j]], add=True)
            fire(j + 2, rows_a, sem_a)
            wait(rows_b, sem_b)
            pltpu.sync_copy(rows_b, acc.at[dst_v.at[j + 1]], add=True)

        wait(rows_a, sem_a)
        pltpu.sync_copy(rows_a, acc.at[dst_v.at[NCHUNK - 1]], add=True)

        plsc.subcore_barrier()
        pltpu.sync_copy(
            acc.at[pl.ds(s * RPS, RPS)], out_hbm.at[c, pl.ds(s * RPS, RPS)]
        )

    return k(u, src3, dst3, zeros)


def _first_layer(x, W1, deg2):
    def body(deg2_ref, x_ref, w_ref, g_ref, u_ref, dinv_ref, dinv2_ref):
        deg = deg2_ref[0, 0:N, 0:1] + deg2_ref[1, 0:N, 0:1] + 1.0
        dinv = lax.rsqrt(deg)
        dinv_ref[...] = dinv
        dinv2_ref[...] = dinv * dinv
        g = jnp.dot(x_ref[...], w_ref[...], preferred_element_type=jnp.float32)
        g_ref[...] = g
        u_ref[...] = dinv * g

    dout = W1.shape[1]
    return pl.pallas_call(
        body,
        out_shape=(
            jax.ShapeDtypeStruct((N, dout), jnp.float32),
            jax.ShapeDtypeStruct((N, dout), jnp.float32),
            jax.ShapeDtypeStruct((N, 1), jnp.float32),
            jax.ShapeDtypeStruct((N, 1), jnp.float32),
        ),
    )(deg2, x, W1)


def _mid_layer(s2, g, dinv, dinv2, b, W, relu, pad_u_to=None):
    def body(s2_ref, g_ref, dinv_ref, dinv2_ref, b_ref, w_ref, g_out, u_out):
        pre = (
            dinv_ref[...] * (s2_ref[0, 0:N] + s2_ref[1, 0:N])
            + dinv2_ref[...] * g_ref[...]
            + b_ref[...]
        )
        h = jnp.maximum(pre, 0.0) if relu else pre
        gn = jnp.dot(h, w_ref[...], preferred_element_type=jnp.float32)
        g_out[...] = gn
        un = dinv_ref[...] * gn
        if pad_u_to is None:
            u_out[...] = un
        else:
            col = lax.broadcasted_iota(jnp.int32, (N, pad_u_to), 1)
            u_out[...] = jnp.where(col == 0, un, 0.0)

    dout = W.shape[1]
    du = dout if pad_u_to is None else pad_u_to
    return pl.pallas_call(
        body,
        out_shape=(
            jax.ShapeDtypeStruct((N, dout), jnp.float32),
            jax.ShapeDtypeStruct((N, du), jnp.float32),
        ),
    )(s2, g, dinv, dinv2, b.reshape(1, W.shape[0]), W)


def _final_layer(s2, g, dinv, dinv2, b):
    def body(s2_ref, g_ref, dinv_ref, dinv2_ref, b_ref, o_ref):
        srow = s2_ref[0, 0:N, 0:1] + s2_ref[1, 0:N, 0:1]
        o_ref[...] = dinv_ref[...] * srow + dinv2_ref[...] * g_ref[...] + b_ref[...]

    return pl.pallas_call(
        body,
        out_shape=jax.ShapeDtypeStruct((N, 1), jnp.float32),
    )(s2, g, dinv, dinv2, b.reshape(1, 1))


def kernel(x, edge_index, W1, W2, W3, W4, W5, W6, W7, W8, b1, b2, b3, b4, b5, b6, b7, b8):
    src3 = edge_index[0].reshape(NW, NCHUNK, G)
    dst3 = edge_index[1].reshape(NW, NCHUNK, G)

    ones16 = jnp.ones((N, 16), jnp.float32)
    zeros16 = jnp.zeros((RPS, 16), jnp.float32)

    deg2 = _propagate(ones16, src3, dst3, zeros16, 16)
    g, u, dinv, dinv2 = _first_layer(x, W1, deg2)

    Ws = [W2, W3, W4, W5, W6, W7, W8]
    bs = [b1, b2, b3, b4, b5, b6, b7]
    relus = [True, True, True, False, True, True, True]
    for i in range(7):
        d = u.shape[1]
        s2 = _propagate(u, src3, dst3, jnp.zeros((RPS, d), jnp.float32), d)
        pad = 16 if i == 6 else None
        g, u = _mid_layer(s2, g, dinv, dinv2, bs[i], Ws[i], relus[i], pad_u_to=pad)

    s2 = _propagate(u, src3, dst3, zeros16, 16)
    return _final_layer(s2, g, dinv, dinv2, b8)


# speedup vs baseline: 24.7876x; 1.1673x over previous
import functools

import jax
import jax.numpy as jnp
from jax import lax
from jax.experimental import pallas as pl
from jax.experimental.pallas import tpu as pltpu
from jax.experimental.pallas import tpu_sc as plsc

N = 10000
E = 320000


def _chunking(d):
    g_ = 40 if d >= 128 else 80
    return g_, EPW // g_
NC = 2
NS = 16
NW = NC * NS
EPW = E // NW
G = 80
NCHUNK = EPW // G
NP = 10112
RPS = NP // NS


def _propagate(u, src3, dst3, zeros, d):
    g_, nchunk = _chunking(d)
    mesh = plsc.VectorSubcoreMesh(core_axis_name="c", subcore_axis_name="s")

    @functools.partial(
        pl.kernel,
        mesh=mesh,
        out_type=jax.ShapeDtypeStruct((NC, NP, d), jnp.float32),
        scratch_types=[
            pltpu.VMEM((nchunk, g_), jnp.int32),
            pltpu.VMEM((nchunk, g_), jnp.int32),
            pltpu.VMEM((g_, d), jnp.float32),
            pltpu.VMEM((g_, d), jnp.float32),
            pltpu.VMEM((g_, d), jnp.float32),
            pltpu.VMEM((g_, d), jnp.float32),
            pltpu.VMEM_SHARED((NP, d), jnp.float32),
            [pltpu.SemaphoreType.DMA] * 4,
            [pltpu.SemaphoreType.DMA] * 4,
        ],
        compiler_params=pltpu.CompilerParams(use_tc_tiling_on_sc=False),
    )
    def k(u_hbm, src_hbm, dst_hbm, zero_hbm, out_hbm, src_v, dst_v, b0, b1, b2, b3, acc, gsems, ssems):
        c = lax.axis_index("c")
        s = lax.axis_index("s")
        wid = c * NS + s
        bufs = [b0, b1, b2, b3]

        pltpu.sync_copy(zero_hbm, acc.at[pl.ds(s * RPS, RPS)])
        pltpu.sync_copy(src_hbm.at[wid], src_v)
        pltpu.sync_copy(dst_hbm.at[wid], dst_v)
        plsc.subcore_barrier()

        def fire_g(j, k4):
            pltpu.async_copy(u_hbm.at[src_v.at[j]], bufs[k4], gsems[k4])

        def wait_g(k4):
            pltpu.make_async_copy(u_hbm.at[src_v.at[0]], bufs[k4], gsems[k4]).wait()

        def fire_s(j, k4):
            pltpu.async_copy(bufs[k4], acc.at[dst_v.at[j]], ssems[k4], add=True)

        def wait_s(k4):
            pltpu.make_async_copy(bufs[k4], acc.at[dst_v.at[0]], ssems[k4]).wait()

        def step(j, k4, do_wait_s=True, do_fire_g=True):
            n4 = (k4 + 2) % 4
            if do_wait_s:
                wait_s(n4)
            if do_fire_g:
                fire_g(j + 2, n4)
            wait_g(k4)
            fire_s(j, k4)

        fire_g(0, 0)
        fire_g(1, 1)
        step(0, 0, do_wait_s=False)
        step(1, 1, do_wait_s=False)
        step(2, 2)
        step(3, 3)

        steady_hi = (nchunk - 2) // 4

        @pl.loop(1, steady_hi)
        def _(i):
            j0 = 4 * i
            step(j0 + 0, 0)
            step(j0 + 1, 1)
            step(j0 + 2, 2)
            step(j0 + 3, 3)

        for j in range(4 * steady_hi, nchunk):
            step(j, j % 4, do_fire_g=(j + 2 <= nchunk - 1))
        wait_s((nchunk - 2) % 4)
        wait_s((nchunk - 1) % 4)

        plsc.subcore_barrier()
        pltpu.sync_copy(
            acc.at[pl.ds(s * RPS, RPS)], out_hbm.at[c, pl.ds(s * RPS, RPS)]
        )

    return k(u, src3, dst3, zeros)


def _first_layer(x, W1, deg2):
    def body(deg2_ref, x_ref, w_ref, g_ref, u_ref, dinv_ref, dinv2_ref):
        deg = deg2_ref[0, 0:N, 0:1] + deg2_ref[1, 0:N, 0:1] + 1.0
        dinv = lax.rsqrt(deg)
        dinv_ref[...] = dinv
        dinv2_ref[...] = dinv * dinv
        g = jnp.dot(x_ref[...], w_ref[...], preferred_element_type=jnp.float32)
        g_ref[...] = g
        u_ref[...] = dinv * g

    dout = W1.shape[1]
    return pl.pallas_call(
        body,
        out_shape=(
            jax.ShapeDtypeStruct((N, dout), jnp.float32),
            jax.ShapeDtypeStruct((N, dout), jnp.float32),
            jax.ShapeDtypeStruct((N, 1), jnp.float32),
            jax.ShapeDtypeStruct((N, 1), jnp.float32),
        ),
    )(deg2, x, W1)


def _mid_layer(s2, g, dinv, dinv2, b, W, relu, pad_u_to=None):
    def body(s2_ref, g_ref, dinv_ref, dinv2_ref, b_ref, w_ref, g_out, u_out):
        pre = (
            dinv_ref[...] * (s2_ref[0, 0:N] + s2_ref[1, 0:N])
            + dinv2_ref[...] * g_ref[...]
            + b_ref[...]
        )
        h = jnp.maximum(pre, 0.0) if relu else pre
        gn = jnp.dot(h, w_ref[...], preferred_element_type=jnp.float32)
        g_out[...] = gn
        un = dinv_ref[...] * gn
        if pad_u_to is None:
            u_out[...] = un
        else:
            col = lax.broadcasted_iota(jnp.int32, (N, pad_u_to), 1)
            u_out[...] = jnp.where(col == 0, un, 0.0)

    dout = W.shape[1]
    du = dout if pad_u_to is None else pad_u_to
    return pl.pallas_call(
        body,
        out_shape=(
            jax.ShapeDtypeStruct((N, dout), jnp.float32),
            jax.ShapeDtypeStruct((N, du), jnp.float32),
        ),
    )(s2, g, dinv, dinv2, b.reshape(1, W.shape[0]), W)


def _final_layer(s2, g, dinv, dinv2, b):
    def body(s2_ref, g_ref, dinv_ref, dinv2_ref, b_ref, o_ref):
        srow = s2_ref[0, 0:N, 0:1] + s2_ref[1, 0:N, 0:1]
        o_ref[...] = dinv_ref[...] * srow + dinv2_ref[...] * g_ref[...] + b_ref[...]

    return pl.pallas_call(
        body,
        out_shape=jax.ShapeDtypeStruct((N, 1), jnp.float32),
    )(s2, g, dinv, dinv2, b.reshape(1, 1))


def kernel(x, edge_index, W1, W2, W3, W4, W5, W6, W7, W8, b1, b2, b3, b4, b5, b6, b7, b8):
    def edges(d):
        g_, nchunk = _chunking(d)
        return (edge_index[0].reshape(NW, nchunk, g_),
                edge_index[1].reshape(NW, nchunk, g_))

    ones16 = jnp.ones((N, 16), jnp.float32)
    zeros16 = jnp.zeros((RPS, 16), jnp.float32)

    deg2 = _propagate(ones16, *edges(16), zeros16, 16)
    g, u, dinv, dinv2 = _first_layer(x, W1, deg2)

    Ws = [W2, W3, W4, W5, W6, W7, W8]
    bs = [b1, b2, b3, b4, b5, b6, b7]
    relus = [True, True, True, False, True, True, True]
    for i in range(7):
        d = u.shape[1]
        s2 = _propagate(u, *edges(d), jnp.zeros((RPS, d), jnp.float32), d)
        pad = 16 if i == 6 else None
        g, u = _mid_layer(s2, g, dinv, dinv2, bs[i], Ws[i], relus[i], pad_u_to=pad)

    s2 = _propagate(u, *edges(16), zeros16, 16)
    return _final_layer(s2, g, dinv, dinv2, b8)


# speedup vs baseline: 26.4762x; 1.0681x over previous
import functools

import jax
import jax.numpy as jnp
from jax import lax
from jax.experimental import pallas as pl
from jax.experimental.pallas import tpu as pltpu
from jax.experimental.pallas import tpu_sc as plsc

N = 10000
E = 320000
NC = 2
NS = 16
NW = NC * NS
EPW = E // NW
NP = 10112
RPS = NP // NS


def _chunking(d):
    g_ = 40 if d >= 128 else 80
    return g_, EPW // g_


_MESH = plsc.VectorSubcoreMesh(core_axis_name="c", subcore_axis_name="s")
_SC_PARAMS = pltpu.CompilerParams(use_tc_tiling_on_sc=False)


def _propagate(u, src3, dst3, zeros, d):
    g_, nchunk = _chunking(d)

    @functools.partial(
        pl.kernel,
        mesh=_MESH,
        out_type=jax.ShapeDtypeStruct((NC, NP, d), jnp.float32),
        scratch_types=[
            pltpu.VMEM((nchunk, g_), jnp.int32),
            pltpu.VMEM((nchunk, g_), jnp.int32),
            pltpu.VMEM((g_, d), jnp.float32),
            pltpu.VMEM((g_, d), jnp.float32),
            pltpu.VMEM((g_, d), jnp.float32),
            pltpu.VMEM((g_, d), jnp.float32),
            pltpu.VMEM_SHARED((NP, d), jnp.float32),
            [pltpu.SemaphoreType.DMA] * 4,
            [pltpu.SemaphoreType.DMA] * 4,
        ],
        compiler_params=_SC_PARAMS,
    )
    def k(u_hbm, src_hbm, dst_hbm, zero_hbm, out_hbm, src_v, dst_v, b0, b1, b2, b3, acc, gsems, ssems):
        c = lax.axis_index("c")
        s = lax.axis_index("s")
        wid = c * NS + s
        bufs = [b0, b1, b2, b3]

        pltpu.sync_copy(zero_hbm, acc.at[pl.ds(s * RPS, RPS)])
        pltpu.sync_copy(src_hbm.at[wid], src_v)
        pltpu.sync_copy(dst_hbm.at[wid], dst_v)
        plsc.subcore_barrier()

        def fire_g(j, k4):
            pltpu.async_copy(u_hbm.at[src_v.at[j]], bufs[k4], gsems[k4])

        def wait_g(k4):
            pltpu.make_async_copy(u_hbm.at[src_v.at[0]], bufs[k4], gsems[k4]).wait()

        def fire_s(j, k4):
            pltpu.async_copy(bufs[k4], acc.at[dst_v.at[j]], ssems[k4], add=True)

        def wait_s(k4):
            pltpu.make_async_copy(bufs[k4], acc.at[dst_v.at[0]], ssems[k4]).wait()

        def step(j, k4, do_wait_s=True, do_fire_g=True):
            n4 = (k4 + 2) % 4
            if do_wait_s:
                wait_s(n4)
            if do_fire_g:
                fire_g(j + 2, n4)
            wait_g(k4)
            fire_s(j, k4)

        fire_g(0, 0)
        fire_g(1, 1)
        step(0, 0, do_wait_s=False)
        step(1, 1, do_wait_s=False)
        step(2, 2)
        step(3, 3)

        steady_hi = (nchunk - 2) // 4

        @pl.loop(1, steady_hi)
        def _(i):
            j0 = 4 * i
            step(j0 + 0, 0)
            step(j0 + 1, 1)
            step(j0 + 2, 2)
            step(j0 + 3, 3)

        for j in range(4 * steady_hi, nchunk):
            step(j, j % 4, do_fire_g=(j + 2 <= nchunk - 1))
        wait_s((nchunk - 2) % 4)
        wait_s((nchunk - 1) % 4)

        plsc.subcore_barrier()
        pltpu.sync_copy(
            acc.at[pl.ds(s * RPS, RPS)], out_hbm.at[c, pl.ds(s * RPS, RPS)]
        )

    return k(u, src3, dst3, zeros)


def _degree(dst3, zeros, d=16):
    g_, nchunk = _chunking(d)

    @functools.partial(
        pl.kernel,
        mesh=_MESH,
        out_type=jax.ShapeDtypeStruct((NC, NP, d), jnp.float32),
        scratch_types=[
            pltpu.VMEM((nchunk, g_), jnp.int32),
            pltpu.VMEM((g_, d), jnp.float32),
            pltpu.VMEM((g_, d), jnp.float32),
            pltpu.VMEM((g_, d), jnp.float32),
            pltpu.VMEM((g_, d), jnp.float32),
            pltpu.VMEM_SHARED((NP, d), jnp.float32),
            [pltpu.SemaphoreType.DMA] * 4,
        ],
        compiler_params=_SC_PARAMS,
    )
    def k(dst_hbm, zero_hbm, out_hbm, dst_v, b0, b1, b2, b3, acc, ssems):
        c = lax.axis_index("c")
        s = lax.axis_index("s")
        wid = c * NS + s
        bufs = [b0, b1, b2, b3]

        pltpu.sync_copy(zero_hbm, acc.at[pl.ds(s * RPS, RPS)])
        pltpu.sync_copy(dst_hbm.at[wid], dst_v)
        one = jnp.full((16,), 1.0, dtype=jnp.float32)

        @pl.loop(0, g_)
        def _(r):
            b0[r, :] = one
            b1[r, :] = one
            b2[r, :] = one
            b3[r, :] = one

        plsc.subcore_barrier()

        def fire_s(j, k4):
            pltpu.async_copy(bufs[k4], acc.at[dst_v.at[j]], ssems[k4], add=True)

        def wait_s(k4):
            pltpu.make_async_copy(bufs[k4], acc.at[dst_v.at[0]], ssems[k4]).wait()

        for j in range(4):
            fire_s(j, j)

        @pl.loop(1, nchunk // 4)
        def _(i):
            j0 = 4 * i
            for k4 in range(4):
                wait_s(k4)
                fire_s(j0 + k4, k4)

        for j in range(4 * (nchunk // 4), nchunk):
            wait_s(j % 4)
            fire_s(j, j % 4)
        for k4 in range(4):
            wait_s((nchunk - 4 + k4) % 4)

        plsc.subcore_barrier()
        pltpu.sync_copy(
            acc.at[pl.ds(s * RPS, RPS)], out_hbm.at[c, pl.ds(s * RPS, RPS)]
        )

    return k(dst3, zeros)


def _first_layer(x, W1, deg2):
    def body(deg2_ref, x_ref, w_ref, g_ref, u_ref, dinv_ref, dinv2_ref):
        deg = deg2_ref[0, 0:N, 0:1] + deg2_ref[1, 0:N, 0:1] + 1.0
        dinv = lax.rsqrt(deg)
        dinv_ref[...] = dinv
        dinv2_ref[...] = dinv * dinv
        g = jnp.dot(x_ref[...], w_ref[...], preferred_element_type=jnp.float32)
        g_ref[...] = g
        u_ref[...] = dinv * g

    dout = W1.shape[1]
    return pl.pallas_call(
        body,
        out_shape=(
            jax.ShapeDtypeStruct((N, dout), jnp.float32),
            jax.ShapeDtypeStruct((N, dout), jnp.float32),
            jax.ShapeDtypeStruct((N, 1), jnp.float32),
            jax.ShapeDtypeStruct((N, 1), jnp.float32),
        ),
    )(deg2, x, W1)


def _a_end(s2, g, dinv, dinv2, b):
    def body(s2_ref, g_ref, dinv_ref, dinv2_ref, b_ref, h_out, q_out):
        pre = (
            dinv_ref[...] * (s2_ref[0, 0:N] + s2_ref[1, 0:N])
            + dinv2_ref[...] * g_ref[...]
            + b_ref[...]
        )
        h = jnp.maximum(pre, 0.0)
        h_out[...] = h
        q_out[...] = dinv_ref[...] * h

    d = g.shape[1]
    return pl.pallas_call(
        body,
        out_shape=(
            jax.ShapeDtypeStruct((N, d), jnp.float32),
            jax.ShapeDtypeStruct((N, d), jnp.float32),
        ),
    )(s2, g, dinv, dinv2, b.reshape(1, d))


def _b_mid(s2, h, dinv, dinv2, b, W):
    def body(s2_ref, h_ref, dinv_ref, dinv2_ref, b_ref, w_ref, h_out, q_out):
        ph = (
            dinv_ref[...] * (s2_ref[0, 0:N] + s2_ref[1, 0:N])
            + dinv2_ref[...] * h_ref[...]
        )
        hn = jnp.maximum(
            jnp.dot(ph, w_ref[...], preferred_element_type=jnp.float32)
            + b_ref[...],
            0.0,
        )
        h_out[...] = hn
        q_out[...] = dinv_ref[...] * hn

    dout = W.shape[1]
    return pl.pallas_call(
        body,
        out_shape=(
            jax.ShapeDtypeStruct((N, dout), jnp.float32),
            jax.ShapeDtypeStruct((N, dout), jnp.float32),
        ),
    )(s2, h, dinv, dinv2, b.reshape(1, dout), W)


def _b_to_a(s2, h, dinv, dinv2, b3, W3, W4):
    def body(s2_ref, h_ref, dinv_ref, dinv2_ref, b_ref, w3_ref, w4_ref, g_out, u_out):
        ph = (
            dinv_ref[...] * (s2_ref[0, 0:N] + s2_ref[1, 0:N])
            + dinv2_ref[...] * h_ref[...]
        )
        h4 = jnp.maximum(
            jnp.dot(ph, w3_ref[...], preferred_element_type=jnp.float32)
            + b_ref[...],
            0.0,
        )
        g4 = jnp.dot(h4, w4_ref[...], preferred_element_type=jnp.float32)
        g_out[...] = g4
        u_out[...] = dinv_ref[...] * g4

    dout = W4.shape[1]
    return pl.pallas_call(
        body,
        out_shape=(
            jax.ShapeDtypeStruct((N, dout), jnp.float32),
            jax.ShapeDtypeStruct((N, dout), jnp.float32),
        ),
    )(s2, h, dinv, dinv2, b3.reshape(1, W3.shape[1]), W3, W4)


def _mid_layer(s2, g, dinv, dinv2, b, W, relu, pad_u_to=None):
    def body(s2_ref, g_ref, dinv_ref, dinv2_ref, b_ref, w_ref, g_out, u_out):
        pre = (
            dinv_ref[...] * (s2_ref[0, 0:N] + s2_ref[1, 0:N])
            + dinv2_ref[...] * g_ref[...]
            + b_ref[...]
        )
        h = jnp.maximum(pre, 0.0) if relu else pre
        gn = jnp.dot(h, w_ref[...], preferred_element_type=jnp.float32)
        g_out[...] = gn
        un = dinv_ref[...] * gn
        if pad_u_to is None:
            u_out[...] = un
        else:
            col = lax.broadcasted_iota(jnp.int32, (N, pad_u_to), 1)
            u_out[...] = jnp.where(col == 0, un, 0.0)

    dout = W.shape[1]
    du = dout if pad_u_to is None else pad_u_to
    return pl.pallas_call(
        body,
        out_shape=(
            jax.ShapeDtypeStruct((N, dout), jnp.float32),
            jax.ShapeDtypeStruct((N, du), jnp.float32),
        ),
    )(s2, g, dinv, dinv2, b.reshape(1, W.shape[0]), W)


def _final_layer(s2, g, dinv, dinv2, b):
    def body(s2_ref, g_ref, dinv_ref, dinv2_ref, b_ref, o_ref):
        srow = s2_ref[0, 0:N, 0:1] + s2_ref[1, 0:N, 0:1]
        o_ref[...] = dinv_ref[...] * srow + dinv2_ref[...] * g_ref[...] + b_ref[...]

    return pl.pallas_call(
        body,
        out_shape=jax.ShapeDtypeStruct((N, 1), jnp.float32),
    )(s2, g, dinv, dinv2, b.reshape(1, 1))


def kernel(x, edge_index, W1, W2, W3, W4, W5, W6, W7, W8, b1, b2, b3, b4, b5, b6, b7, b8):
    def edges(d):
        g_, nchunk = _chunking(d)
        return (edge_index[0].reshape(NW, nchunk, g_),
                edge_index[1].reshape(NW, nchunk, g_))

    def zeros(d):
        return jnp.zeros((RPS, d), jnp.float32)

    def prop(u, d):
        return _propagate(u, *edges(d), zeros(d), d)

    deg2 = _degree(edges(16)[1], zeros(16))
    g1, u1, dinv, dinv2 = _first_layer(x, W1, deg2)
    s1 = prop(u1, 64)
    h2, q2 = _a_end(s1, g1, dinv, dinv2, b1)
    s2 = prop(q2, 64)
    h3, q3 = _b_mid(s2, h2, dinv, dinv2, b2, W2)
    s3 = prop(q3, 96)
    g4, u4 = _b_to_a(s3, h3, dinv, dinv2, b3, W3, W4)
    s4 = prop(u4, 128)
    g5, u5 = _mid_layer(s4, g4, dinv, dinv2, b4, W5, relu=False)
    s5 = prop(u5, 128)
    g6, u6 = _mid_layer(s5, g5, dinv, dinv2, b5, W6, relu=True)
    s6 = prop(u6, 96)
    g7, u7 = _mid_layer(s6, g6, dinv, dinv2, b6, W7, relu=True)
    s7 = prop(u7, 64)
    g8, u8 = _mid_layer(s7, g7, dinv, dinv2, b7, W8, relu=True, pad_u_to=16)
    s8 = prop(u8, 16)
    return _final_layer(s8, g8, dinv, dinv2, b8)
